# Initial kernel scaffold; baseline (speedup 1.0000x reference)
import jax
import jax.numpy as jnp
from jax.experimental import pallas as pl


def kernel(boxes, scores):
    raise NotImplementedError("write your pallas kernel here")


# speedup vs baseline: 138.0697x; 138.0697x over previous
import jax
import jax.numpy as jnp
from jax import lax
from jax.experimental import pallas as pl
from jax.experimental.pallas import tpu as pltpu
from jax.experimental.pallas import tpu_sc as plsc

NMS_T = 0.5
SCORE_T = 0.05
DETS = 100
NEG = -1e9
BIGIDX = float(2 ** 30)
NTILES = 1
LANES = 16

_axis_index = lax.axis_index


def _nms_body(x1h, y1h, x2h, y2h, sh, outh,
              x1v, y1v, x2v, y2v, sv, av, rec, allrec, outv, selflag, cand):
    chunk = x1v.shape[0]
    vecs = chunk // LANES
    c = _axis_index("c")
    t = _axis_index("s")
    multi = NTILES > 1
    active = (c == 0) & (t < NTILES) if multi else (c == 0) & (t == 0)

    @pl.when(active)
    def _work():
        base = t * chunk
        pltpu.sync_copy(x1h.at[pl.ds(base, chunk)], x1v)
        pltpu.sync_copy(y1h.at[pl.ds(base, chunk)], y1v)
        pltpu.sync_copy(x2h.at[pl.ds(base, chunk)], x2v)
        pltpu.sync_copy(y2h.at[pl.ds(base, chunk)], y2v)
        pltpu.sync_copy(sh.at[pl.ds(base, chunk)], sv)

        def _areas(j, carry):
            sl = pl.ds(j * LANES, LANES)
            av[sl] = (x2v[sl] - x1v[sl]) * (y2v[sl] - y1v[sl])
            return carry
        lax.fori_loop(0, vecs, _areas, 0)

        @pl.when(t == 0)
        def _zero():
            def _z(j, carry):
                selflag[pl.ds(j * LANES, LANES)] = jnp.zeros((LANES,), jnp.int32)
                return carry
            lax.fori_loop(0, 256 // LANES, _z, 0)

        lanes = lax.iota(jnp.int32, LANES)

        def _iter(k, m):
            def _mx(j, acc):
                return jnp.maximum(acc, sv[pl.ds(j * LANES, LANES)])
            mvec = lax.fori_loop(0, vecs, _mx,
                                 jnp.full((LANES,), -jnp.inf, jnp.float32))
            mval = jnp.max(mvec)

            def _mi(j, acc):
                sj = sv[pl.ds(j * LANES, LANES)]
                gi = (base + j * LANES + lanes).astype(jnp.float32)
                return jnp.minimum(acc, jnp.where(sj == mval, gi, BIGIDX))
            ivec = lax.fori_loop(0, vecs, _mi,
                                 jnp.full((LANES,), BIGIDX, jnp.float32))
            gidxf = jnp.min(ivec)
            lidx = gidxf.astype(jnp.int32) - base
            lsplat = jnp.full((LANES,), lidx, jnp.int32)

            bx1 = plsc.load_gather(x1v, [lsplat])
            by1 = plsc.load_gather(y1v, [lsplat])
            bx2 = plsc.load_gather(x2v, [lsplat])
            by2 = plsc.load_gather(y2v, [lsplat])
            ba = plsc.load_gather(av, [lsplat])
            rvec = jnp.where(lanes == 0, bx1,
                   jnp.where(lanes == 1, by1,
                   jnp.where(lanes == 2, bx2,
                   jnp.where(lanes == 3, by2,
                   jnp.where(lanes == 4, mval,
                   jnp.where(lanes == 5, gidxf, ba))))))

            if multi:
                rec[...] = rvec
                pltpu.sync_copy(rec, cand.at[t])
                plsc.subcore_barrier()
                pltpu.sync_copy(cand, allrec)
                plsc.subcore_barrier()

                svec = plsc.load_gather(
                    allrec, [lanes, jnp.full((LANES,), 4, jnp.int32)])
                ivec2 = plsc.load_gather(
                    allrec, [lanes, jnp.full((LANES,), 5, jnp.int32)])
                wval = jnp.max(svec)
                wsel = svec == wval
                widxf = jnp.min(jnp.where(wsel, ivec2, BIGIDX))
                wmask = wsel & (ivec2 == widxf)
                wtile = jnp.min(jnp.where(wmask, lanes, jnp.int32(LANES - 1)))
                wrec = plsc.load_gather(
                    allrec, [jnp.full((LANES,), wtile, jnp.int32), lanes])
            else:
                wval = mval
                widxf = gidxf
                wrec = rvec

            wx1 = wrec[0]
            wy1 = wrec[1]
            wx2 = wrec[2]
            wy2 = wrec[3]
            wa = wrec[6]

            valid = wval > SCORE_T

            @pl.when((t == 0) & valid)
            def _emit():
                outv[pl.ds(m * LANES, LANES)] = wrec
                plsc.store_scatter(
                    selflag, [jnp.full((LANES,), widxf.astype(jnp.int32),
                                       jnp.int32)],
                    jnp.ones((LANES,), jnp.int32), mask=lanes == 0)

            def _sup(j, carry):
                sl = pl.ds(j * LANES, LANES)
                xx1 = jnp.maximum(wx1, x1v[sl])
                yy1 = jnp.maximum(wy1, y1v[sl])
                xx2 = jnp.minimum(wx2, x2v[sl])
                yy2 = jnp.minimum(wy2, y2v[sl])
                inter = (jnp.maximum(xx2 - xx1, 0.0) *
                         jnp.maximum(yy2 - yy1, 0.0))
                iou = inter / (wa + av[sl] - inter + 1e-9)
                sv[sl] = jnp.where(iou > NMS_T, NEG, sv[sl])
                return carry
            lax.fori_loop(0, vecs, _sup, 0)

            return m + jnp.where(valid, jnp.int32(1), jnp.int32(0))

        m = lax.fori_loop(0, DETS, _iter, jnp.int32(0))

        @pl.when(t == 0)
        def _finish():
            def _fb(j, run):
                sl = pl.ds(j * LANES, LANES)
                z = selflag[sl] == 0
                inc = z.astype(jnp.int32)
                cum = plsc.cumsum(inc)
                slot = m + run + (cum - inc)
                en = z & (slot < DETS)
                sbase = slot * LANES
                plsc.store_scatter(outv, [sbase + 0], x1v[sl], mask=en)
                plsc.store_scatter(outv, [sbase + 1], y1v[sl], mask=en)
                plsc.store_scatter(outv, [sbase + 2], x2v[sl], mask=en)
                plsc.store_scatter(outv, [sbase + 3], y2v[sl], mask=en)
                plsc.store_scatter(outv, [sbase + 4],
                                   jnp.full((LANES,), NEG, jnp.float32),
                                   mask=en)
                return run + cum[LANES - 1]
            lax.fori_loop(0, 256 // LANES, _fb, jnp.int32(0))
            pltpu.sync_copy(outv, outh)


def _build(n, interpret=False):
    chunk = -(-n // (NTILES * LANES)) * LANES
    npad = chunk * NTILES
    mesh = plsc.VectorSubcoreMesh(
        core_axis_name="c", subcore_axis_name="s", num_cores=1)
    f = pl.kernel(
        _nms_body,
        out_type=jax.ShapeDtypeStruct((DETS * LANES,), jnp.float32),
        mesh=mesh,
        compiler_params=pltpu.CompilerParams(needs_layout_passes=False),
        interpret=interpret,
        scratch_types=[
            pltpu.VMEM((chunk,), jnp.float32),
            pltpu.VMEM((chunk,), jnp.float32),
            pltpu.VMEM((chunk,), jnp.float32),
            pltpu.VMEM((chunk,), jnp.float32),
            pltpu.VMEM((chunk,), jnp.float32),
            pltpu.VMEM((chunk,), jnp.float32),
            pltpu.VMEM((LANES,), jnp.float32),
            pltpu.VMEM((NTILES, LANES), jnp.float32),
            pltpu.VMEM((DETS * LANES,), jnp.float32),
            pltpu.VMEM((npad,), jnp.int32),
            pltpu.VMEM_SHARED((NTILES, LANES), jnp.float32),
        ],
    )
    return f, npad


def kernel(boxes, scores):
    n = boxes.shape[0]
    f, npad = _build(n)
    x1 = jnp.pad(boxes[:, 0], (0, npad - n))
    y1 = jnp.pad(boxes[:, 1], (0, npad - n))
    x2 = jnp.pad(boxes[:, 2], (0, npad - n))
    y2 = jnp.pad(boxes[:, 3], (0, npad - n))
    sp = jnp.pad(scores, (0, npad - n), constant_values=NEG)
    out = f(x1, y1, x2, y2, sp)
    return out.reshape(DETS, LANES)[:, :5]


# speedup vs baseline: 342.7141x; 2.4822x over previous
import jax
import jax.numpy as jnp
from jax import lax
from jax.experimental import pallas as pl
from jax.experimental.pallas import tpu as pltpu
from jax.experimental.pallas import tpu_sc as plsc

NMS_T = 0.5
SCORE_T = 0.05
DETS = 100
NEG = -1e9
BIGIDX = float(2 ** 30)
LANES = 16
UNROLL = 4

_axis_index = lax.axis_index


def _nms_body(x1h, y1h, x2h, y2h, sh, outh,
              x1v, y1v, x2v, y2v, sv, av, outv, selflag):
    chunk = x1v.shape[0]
    vecs = chunk // LANES
    c = _axis_index("c")
    t = _axis_index("s")

    @pl.when((c == 0) & (t == 0))
    def _work():
        pltpu.sync_copy(x1h, x1v)
        pltpu.sync_copy(y1h, y1v)
        pltpu.sync_copy(x2h, x2v)
        pltpu.sync_copy(y2h, y2v)
        pltpu.sync_copy(sh, sv)

        def _areas(j, carry):
            sl = pl.ds(j * LANES, LANES)
            av[sl] = (x2v[sl] - x1v[sl]) * (y2v[sl] - y1v[sl])
            return carry
        lax.fori_loop(0, vecs, _areas, 0)

        def _z(j, carry):
            selflag[pl.ds(j * LANES, LANES)] = jnp.zeros((LANES,), jnp.int32)
            return carry
        lax.fori_loop(0, 256 // LANES, _z, 0)

        lanes = lax.iota(jnp.int32, LANES)
        ninf = jnp.full((LANES,), -jnp.inf, jnp.float32)
        bigv = jnp.full((LANES,), BIGIDX, jnp.float32)

        def _track(sj, gif, acc):
            accv, acci = acc
            take = sj > accv
            return jnp.where(take, sj, accv), jnp.where(take, gif, acci)

        def _reduce_winner(accv, acci):
            mval = jnp.max(accv)
            gidxf = jnp.min(jnp.where(accv == mval, acci, bigv))
            lidx = gidxf.astype(jnp.int32)
            lsplat = jnp.full((LANES,), lidx, jnp.int32)
            bx1 = plsc.load_gather(x1v, [lsplat])
            by1 = plsc.load_gather(y1v, [lsplat])
            bx2 = plsc.load_gather(x2v, [lsplat])
            by2 = plsc.load_gather(y2v, [lsplat])
            ba = plsc.load_gather(av, [lsplat])
            rvec = jnp.where(lanes == 0, bx1,
                   jnp.where(lanes == 1, by1,
                   jnp.where(lanes == 2, bx2,
                   jnp.where(lanes == 3, by2,
                   jnp.where(lanes == 4, mval,
                   jnp.where(lanes == 5, gidxf, ba))))))
            return rvec

        def _emit(m, rvec):
            wval = rvec[4]
            valid = wval > SCORE_T

            @pl.when(valid)
            def _():
                outv[pl.ds(m * LANES, LANES)] = rvec
                plsc.store_scatter(
                    selflag,
                    [jnp.full((LANES,), rvec[5].astype(jnp.int32), jnp.int32)],
                    jnp.ones((LANES,), jnp.int32), mask=lanes == 0)
            return m + jnp.where(valid, jnp.int32(1), jnp.int32(0))

        def _mx0(j, acc):
            accv, acci = acc
            for u in range(UNROLL):
                jj = j * UNROLL + u
                sl = pl.ds(jj * LANES, LANES)
                gif = (jj * LANES + lanes).astype(jnp.float32)
                accv, acci = _track(sv[sl], gif, (accv, acci))
            return accv, acci
        accv, acci = lax.fori_loop(0, vecs // UNROLL, _mx0, (ninf, bigv))
        rvec = _reduce_winner(accv, acci)
        m0 = _emit(jnp.int32(0), rvec)

        def _iter(k, carry):
            m, wrec = carry
            wx1 = wrec[0]
            wy1 = wrec[1]
            wx2 = wrec[2]
            wy2 = wrec[3]
            wa = wrec[6]

            def _fp(j, acc):
                accv, acci = acc
                for u in range(UNROLL):
                    jj = j * UNROLL + u
                    sl = pl.ds(jj * LANES, LANES)
                    xx1 = jnp.maximum(wx1, x1v[sl])
                    yy1 = jnp.maximum(wy1, y1v[sl])
                    xx2 = jnp.minimum(wx2, x2v[sl])
                    yy2 = jnp.minimum(wy2, y2v[sl])
                    inter = (jnp.maximum(xx2 - xx1, 0.0) *
                             jnp.maximum(yy2 - yy1, 0.0))
                    iou = inter / (wa + av[sl] - inter + 1e-9)
                    snew = jnp.where(iou > NMS_T, NEG, sv[sl])
                    sv[sl] = snew
                    gif = (jj * LANES + lanes).astype(jnp.float32)
                    accv, acci = _track(snew, gif, (accv, acci))
                return accv, acci
            accv, acci = lax.fori_loop(0, vecs // UNROLL, _fp, (ninf, bigv))
            rvec = _reduce_winner(accv, acci)
            return _emit(m, rvec), rvec

        m, _ = lax.fori_loop(1, DETS, _iter, (m0, rvec))

        def _fb(j, run):
            sl = pl.ds(j * LANES, LANES)
            z = selflag[sl] == 0
            inc = z.astype(jnp.int32)
            cum = plsc.cumsum(inc)
            slot = m + run + (cum - inc)
            en = z & (slot < DETS)
            sbase = slot * LANES
            plsc.store_scatter(outv, [sbase + 0], x1v[sl], mask=en)
            plsc.store_scatter(outv, [sbase + 1], y1v[sl], mask=en)
            plsc.store_scatter(outv, [sbase + 2], x2v[sl], mask=en)
            plsc.store_scatter(outv, [sbase + 3], y2v[sl], mask=en)
            plsc.store_scatter(outv, [sbase + 4],
                               jnp.full((LANES,), NEG, jnp.float32),
                               mask=en)
            return run + cum[LANES - 1]
        lax.fori_loop(0, 256 // LANES, _fb, jnp.int32(0))
        pltpu.sync_copy(outv, outh)


def _build(n, interpret=False):
    npad = -(-n // (LANES * UNROLL)) * LANES * UNROLL
    mesh = plsc.VectorSubcoreMesh(
        core_axis_name="c", subcore_axis_name="s", num_cores=1)
    f = pl.kernel(
        _nms_body,
        out_type=jax.ShapeDtypeStruct((DETS * LANES,), jnp.float32),
        mesh=mesh,
        compiler_params=pltpu.CompilerParams(needs_layout_passes=False),
        interpret=interpret,
        scratch_types=[
            pltpu.VMEM((npad,), jnp.float32),
            pltpu.VMEM((npad,), jnp.float32),
            pltpu.VMEM((npad,), jnp.float32),
            pltpu.VMEM((npad,), jnp.float32),
            pltpu.VMEM((npad,), jnp.float32),
            pltpu.VMEM((npad,), jnp.float32),
            pltpu.VMEM((DETS * LANES,), jnp.float32),
            pltpu.VMEM((npad,), jnp.int32),
        ],
    )
    return f, npad


def kernel(boxes, scores):
    n = boxes.shape[0]
    f, npad = _build(n)
    x1 = jnp.pad(boxes[:, 0], (0, npad - n))
    y1 = jnp.pad(boxes[:, 1], (0, npad - n))
    x2 = jnp.pad(boxes[:, 2], (0, npad - n))
    y2 = jnp.pad(boxes[:, 3], (0, npad - n))
    sp = jnp.pad(scores, (0, npad - n), constant_values=NEG)
    out = f(x1, y1, x2, y2, sp)
    return out.reshape(DETS, LANES)[:, :5]
